# baseline (speedup 1.0000x reference)
import jax
import jax.numpy as jnp
from jax.experimental import pallas as pl

_N_USERS = 8192
_N_ITEMS = 4096
_D = 64
_IMG_DIM = 4096
_TXT_DIM = 384
_EPS_NOISE = 0.2
_CAT = 0.55

_F32 = jnp.float32
_BF16 = jnp.bfloat16


def _row_l2norm(x, eps=1e-12):
    n = jnp.sqrt(jnp.sum(x * x, axis=1, keepdims=True))
    return x / jnp.clip(n, eps, None)


def _stage_a(img_ref, txt_ref, emb_ref, wimg_ref, bimg_ref, wtxt_ref, btxt_ref,
             out_ref):
    imf = jnp.dot(img_ref[...].astype(_BF16), wimg_ref[...],
                  preferred_element_type=_F32)
    txf = jnp.dot(txt_ref[...].astype(_BF16), wtxt_ref[...],
                  preferred_element_type=_F32)
    out_ref[...] = jnp.concatenate(
        [imf + bimg_ref[...], txf + btxt_ref[...], emb_ref[...]],
        axis=1).astype(_BF16)


def _stage_prop_noise(g_ref, rhs_ref, noise_ref, out_ref, outm_ref):
    prod = jnp.dot(g_ref[...].astype(_BF16), rhs_ref[...],
                   preferred_element_type=_F32)
    raw = prod[:, 2 * _D:]
    noised = raw + jnp.sign(raw) * _row_l2norm(noise_ref[...]) * _EPS_NOISE
    out_ref[...] = jnp.concatenate([prod, noised], axis=1)
    outm_ref[...] = prod.astype(_BF16)


def _stage_user_final(ui_ref, i1n_ref, uemb_ref, u1p_ref,
                      uout_ref, ucl_ref, ug2_ref):
    ug2 = jnp.dot(ui_ref[...].astype(_BF16), i1n_ref[...],
                  preferred_element_type=_F32)
    u1p = u1p_ref[...]
    mix = _CAT * _row_l2norm(u1p[:, :_D]) + _CAT * _row_l2norm(u1p[:, _D:2 * _D])
    mean = (uemb_ref[...] + u1p[:, 3 * _D:] + ug2) * (1.0 / 3.0)
    uout_ref[...] = mean + mix
    ucl_ref[...] = ug2 + mix
    ug2_ref[...] = ug2.astype(_BF16)


def _stage_item_final(iu_ref, ug2_ref, iemb_ref, i1p_ref, iout_ref, icl_ref):
    ig2 = jnp.dot(iu_ref[...].astype(_BF16), ug2_ref[...],
                  preferred_element_type=_F32)
    i1p = i1p_ref[...]
    mix = _CAT * _row_l2norm(i1p[:, :_D]) + _CAT * _row_l2norm(i1p[:, _D:2 * _D])
    mean = (iemb_ref[...] + i1p[:, 3 * _D:] + ig2) * (1.0 / 3.0)
    iout_ref[...] = mean + mix
    icl_ref[...] = ig2 + mix


def kernel(ui_graph, iu_graph, image_feats, text_feats, user_emb, item_emb,
           W_img, b_img, W_txt, b_txt):
    ND = 3 * _D
    NW = 4 * _D

    nkey = jax.random.key(42)
    u_noise = jax.random.uniform(jax.random.fold_in(nkey, 0), (_N_USERS, _D),
                                 dtype=_F32)
    i_noise = jax.random.uniform(jax.random.fold_in(nkey, 1), (_N_ITEMS, _D),
                                 dtype=_F32)

    wimg_t = W_img.T.astype(_BF16)
    wtxt_t = W_txt.T.astype(_BF16)
    bimg = b_img.reshape(1, _D)
    btxt = b_txt.reshape(1, _D)

    bm_a = 1024
    r0 = pl.pallas_call(
        _stage_a,
        grid=(_N_ITEMS // bm_a,),
        in_specs=[
            pl.BlockSpec((bm_a, _IMG_DIM), lambda i: (i, 0)),
            pl.BlockSpec((bm_a, _TXT_DIM), lambda i: (i, 0)),
            pl.BlockSpec((bm_a, _D), lambda i: (i, 0)),
            pl.BlockSpec((_IMG_DIM, _D), lambda i: (0, 0)),
            pl.BlockSpec((1, _D), lambda i: (0, 0)),
            pl.BlockSpec((_TXT_DIM, _D), lambda i: (0, 0)),
            pl.BlockSpec((1, _D), lambda i: (0, 0)),
        ],
        out_specs=pl.BlockSpec((bm_a, ND), lambda i: (i, 0)),
        out_shape=jax.ShapeDtypeStruct((_N_ITEMS, ND), _BF16),
    )(image_feats, text_feats, item_emb, wimg_t, bimg, wtxt_t, btxt)

    bm_b = 1024
    u1p, u1m = pl.pallas_call(
        _stage_prop_noise,
        grid=(_N_USERS // bm_b,),
        in_specs=[
            pl.BlockSpec((bm_b, _N_ITEMS), lambda i: (i, 0)),
            pl.BlockSpec((_N_ITEMS, ND), lambda i: (0, 0)),
            pl.BlockSpec((bm_b, _D), lambda i: (i, 0)),
        ],
        out_specs=[
            pl.BlockSpec((bm_b, NW), lambda i: (i, 0)),
            pl.BlockSpec((bm_b, ND), lambda i: (i, 0)),
        ],
        out_shape=[
            jax.ShapeDtypeStruct((_N_USERS, NW), _F32),
            jax.ShapeDtypeStruct((_N_USERS, ND), _BF16),
        ],
    )(ui_graph, r0, u_noise)

    bm_c = 512
    i1p, i1m = pl.pallas_call(
        _stage_prop_noise,
        grid=(_N_ITEMS // bm_c,),
        in_specs=[
            pl.BlockSpec((bm_c, _N_USERS), lambda i: (i, 0)),
            pl.BlockSpec((_N_USERS, ND), lambda i: (0, 0)),
            pl.BlockSpec((bm_c, _D), lambda i: (i, 0)),
        ],
        out_specs=[
            pl.BlockSpec((bm_c, NW), lambda i: (i, 0)),
            pl.BlockSpec((bm_c, ND), lambda i: (i, 0)),
        ],
        out_shape=[
            jax.ShapeDtypeStruct((_N_ITEMS, NW), _F32),
            jax.ShapeDtypeStruct((_N_ITEMS, ND), _BF16),
        ],
    )(iu_graph, u1m, i_noise)

    i1n = (i1p[:, 3 * _D:]).astype(_BF16)

    bm_d = 1024
    u_out, u_cl_out, u_g2 = pl.pallas_call(
        _stage_user_final,
        grid=(_N_USERS // bm_d,),
        in_specs=[
            pl.BlockSpec((bm_d, _N_ITEMS), lambda i: (i, 0)),
            pl.BlockSpec((_N_ITEMS, _D), lambda i: (0, 0)),
            pl.BlockSpec((bm_d, _D), lambda i: (i, 0)),
            pl.BlockSpec((bm_d, NW), lambda i: (i, 0)),
        ],
        out_specs=[
            pl.BlockSpec((bm_d, _D), lambda i: (i, 0)),
            pl.BlockSpec((bm_d, _D), lambda i: (i, 0)),
            pl.BlockSpec((bm_d, _D), lambda i: (i, 0)),
        ],
        out_shape=[
            jax.ShapeDtypeStruct((_N_USERS, _D), _F32),
            jax.ShapeDtypeStruct((_N_USERS, _D), _F32),
            jax.ShapeDtypeStruct((_N_USERS, _D), _BF16),
        ],
    )(ui_graph, i1n, user_emb, u1p)

    bm_e = 512
    i_out, i_cl_out = pl.pallas_call(
        _stage_item_final,
        grid=(_N_ITEMS // bm_e,),
        in_specs=[
            pl.BlockSpec((bm_e, _N_USERS), lambda i: (i, 0)),
            pl.BlockSpec((_N_USERS, _D), lambda i: (0, 0)),
            pl.BlockSpec((bm_e, _D), lambda i: (i, 0)),
            pl.BlockSpec((bm_e, NW), lambda i: (i, 0)),
        ],
        out_specs=[
            pl.BlockSpec((bm_e, _D), lambda i: (i, 0)),
            pl.BlockSpec((bm_e, _D), lambda i: (i, 0)),
        ],
        out_shape=[
            jax.ShapeDtypeStruct((_N_ITEMS, _D), _F32),
            jax.ShapeDtypeStruct((_N_ITEMS, _D), _F32),
        ],
    )(iu_graph, u_g2, item_emb, i1p)

    image_user_feats = u1p[:, :_D]
    text_user_feats = u1p[:, _D:2 * _D]
    image_item_feats = i1p[:, :_D]
    text_item_feats = i1p[:, _D:2 * _D]

    prompt_user = jnp.zeros((_N_USERS, _D), dtype=_F32)
    prompt_item = jnp.zeros((_N_ITEMS, _D), dtype=_F32)
    gcl_loss = jnp.float32(0.0)

    return (u_out, i_out, image_item_feats, text_item_feats,
            image_user_feats, text_user_feats, u_out, i_out,
            prompt_user, prompt_item, gcl_loss)


# speedup vs baseline: 1.0734x; 1.0734x over previous
import jax
import jax.numpy as jnp
from jax.experimental import pallas as pl

_N_USERS = 8192
_N_ITEMS = 4096
_D = 64
_IMG_DIM = 4096
_TXT_DIM = 384
_EPS_NOISE = 0.2
_CAT = 0.55

_F32 = jnp.float32
_BF16 = jnp.bfloat16


def _row_l2norm(x, eps=1e-12):
    n = jnp.sqrt(jnp.sum(x * x, axis=1, keepdims=True))
    return x / jnp.clip(n, eps, None)


def _stage_a(img_ref, txt_ref, emb_ref, wimg_ref, bimg_ref, wtxt_ref, btxt_ref,
             out_ref):
    imf = jnp.dot(img_ref[...].astype(_BF16), wimg_ref[...],
                  preferred_element_type=_F32)
    txf = jnp.dot(txt_ref[...].astype(_BF16), wtxt_ref[...],
                  preferred_element_type=_F32)
    out_ref[...] = jnp.concatenate(
        [imf + bimg_ref[...], txf + btxt_ref[...], emb_ref[...]],
        axis=1).astype(_BF16)


def _stage_prop_noise(g_ref, rhs_ref, noise_ref,
                      img_ref, txt_ref, emb_ref, outm_ref, embm_ref):
    prod = jnp.dot(g_ref[...].astype(_BF16), rhs_ref[...],
                   preferred_element_type=_F32)
    raw = prod[:, 2 * _D:]
    noised = raw + jnp.sign(raw) * _row_l2norm(noise_ref[...]) * _EPS_NOISE
    img_ref[...] = prod[:, :_D]
    txt_ref[...] = prod[:, _D:2 * _D]
    emb_ref[...] = noised
    outm_ref[...] = prod.astype(_BF16)
    embm_ref[...] = noised.astype(_BF16)


def _stage_user_final(ui_ref, i1n_ref, uemb_ref, imgu_ref, txtu_ref, u1n_ref,
                      uout_ref, ucl_ref, ug2_ref):
    ug2 = jnp.dot(ui_ref[...].astype(_BF16), i1n_ref[...],
                  preferred_element_type=_F32)
    mix = _CAT * _row_l2norm(imgu_ref[...]) + _CAT * _row_l2norm(txtu_ref[...])
    mean = (uemb_ref[...] + u1n_ref[...] + ug2) * (1.0 / 3.0)
    uout_ref[...] = mean + mix
    ucl_ref[...] = ug2 + mix
    ug2_ref[...] = ug2.astype(_BF16)


def _stage_item_final(iu_ref, ug2_ref, iemb_ref, imgi_ref, txti_ref, i1n_ref,
                      iout_ref, icl_ref):
    ig2 = jnp.dot(iu_ref[...].astype(_BF16), ug2_ref[...],
                  preferred_element_type=_F32)
    mix = _CAT * _row_l2norm(imgi_ref[...]) + _CAT * _row_l2norm(txti_ref[...])
    mean = (iemb_ref[...] + i1n_ref[...] + ig2) * (1.0 / 3.0)
    iout_ref[...] = mean + mix
    icl_ref[...] = ig2 + mix


def kernel(ui_graph, iu_graph, image_feats, text_feats, user_emb, item_emb,
           W_img, b_img, W_txt, b_txt):
    ND = 3 * _D

    nkey = jax.random.key(42)
    u_noise = jax.random.uniform(jax.random.fold_in(nkey, 0), (_N_USERS, _D),
                                 dtype=_F32)
    i_noise = jax.random.uniform(jax.random.fold_in(nkey, 1), (_N_ITEMS, _D),
                                 dtype=_F32)

    wimg_t = W_img.T.astype(_BF16)
    wtxt_t = W_txt.T.astype(_BF16)
    bimg = b_img.reshape(1, _D)
    btxt = b_txt.reshape(1, _D)

    def _row(bm):
        return lambda i: (i, 0)

    def _full():
        return lambda i: (0, 0)

    bm_a = 512
    r0 = pl.pallas_call(
        _stage_a,
        grid=(_N_ITEMS // bm_a,),
        in_specs=[
            pl.BlockSpec((bm_a, _IMG_DIM), _row(bm_a)),
            pl.BlockSpec((bm_a, _TXT_DIM), _row(bm_a)),
            pl.BlockSpec((bm_a, _D), _row(bm_a)),
            pl.BlockSpec((_IMG_DIM, _D), _full()),
            pl.BlockSpec((1, _D), _full()),
            pl.BlockSpec((_TXT_DIM, _D), _full()),
            pl.BlockSpec((1, _D), _full()),
        ],
        out_specs=pl.BlockSpec((bm_a, ND), _row(bm_a)),
        out_shape=jax.ShapeDtypeStruct((_N_ITEMS, ND), _BF16),
    )(image_feats, text_feats, item_emb, wimg_t, bimg, wtxt_t, btxt)

    bm_b = 512
    image_user_feats, text_user_feats, u1n, u1m, _u1nm = pl.pallas_call(
        _stage_prop_noise,
        grid=(_N_USERS // bm_b,),
        in_specs=[
            pl.BlockSpec((bm_b, _N_ITEMS), _row(bm_b)),
            pl.BlockSpec((_N_ITEMS, ND), _full()),
            pl.BlockSpec((bm_b, _D), _row(bm_b)),
        ],
        out_specs=[
            pl.BlockSpec((bm_b, _D), _row(bm_b)),
            pl.BlockSpec((bm_b, _D), _row(bm_b)),
            pl.BlockSpec((bm_b, _D), _row(bm_b)),
            pl.BlockSpec((bm_b, ND), _row(bm_b)),
            pl.BlockSpec((bm_b, _D), _row(bm_b)),
        ],
        out_shape=[
            jax.ShapeDtypeStruct((_N_USERS, _D), _F32),
            jax.ShapeDtypeStruct((_N_USERS, _D), _F32),
            jax.ShapeDtypeStruct((_N_USERS, _D), _F32),
            jax.ShapeDtypeStruct((_N_USERS, ND), _BF16),
            jax.ShapeDtypeStruct((_N_USERS, _D), _BF16),
        ],
    )(ui_graph, r0, u_noise)

    bm_c = 256
    image_item_feats, text_item_feats, i1n, _i1m, i1nm = pl.pallas_call(
        _stage_prop_noise,
        grid=(_N_ITEMS // bm_c,),
        in_specs=[
            pl.BlockSpec((bm_c, _N_USERS), _row(bm_c)),
            pl.BlockSpec((_N_USERS, ND), _full()),
            pl.BlockSpec((bm_c, _D), _row(bm_c)),
        ],
        out_specs=[
            pl.BlockSpec((bm_c, _D), _row(bm_c)),
            pl.BlockSpec((bm_c, _D), _row(bm_c)),
            pl.BlockSpec((bm_c, _D), _row(bm_c)),
            pl.BlockSpec((bm_c, ND), _row(bm_c)),
            pl.BlockSpec((bm_c, _D), _row(bm_c)),
        ],
        out_shape=[
            jax.ShapeDtypeStruct((_N_ITEMS, _D), _F32),
            jax.ShapeDtypeStruct((_N_ITEMS, _D), _F32),
            jax.ShapeDtypeStruct((_N_ITEMS, _D), _F32),
            jax.ShapeDtypeStruct((_N_ITEMS, ND), _BF16),
            jax.ShapeDtypeStruct((_N_ITEMS, _D), _BF16),
        ],
    )(iu_graph, u1m, i_noise)

    bm_d = 512
    u_out, u_cl_out, u_g2 = pl.pallas_call(
        _stage_user_final,
        grid=(_N_USERS // bm_d,),
        in_specs=[
            pl.BlockSpec((bm_d, _N_ITEMS), _row(bm_d)),
            pl.BlockSpec((_N_ITEMS, _D), _full()),
            pl.BlockSpec((bm_d, _D), _row(bm_d)),
            pl.BlockSpec((bm_d, _D), _row(bm_d)),
            pl.BlockSpec((bm_d, _D), _row(bm_d)),
            pl.BlockSpec((bm_d, _D), _row(bm_d)),
        ],
        out_specs=[
            pl.BlockSpec((bm_d, _D), _row(bm_d)),
            pl.BlockSpec((bm_d, _D), _row(bm_d)),
            pl.BlockSpec((bm_d, _D), _row(bm_d)),
        ],
        out_shape=[
            jax.ShapeDtypeStruct((_N_USERS, _D), _F32),
            jax.ShapeDtypeStruct((_N_USERS, _D), _F32),
            jax.ShapeDtypeStruct((_N_USERS, _D), _BF16),
        ],
    )(ui_graph, i1nm, user_emb, image_user_feats, text_user_feats, u1n)

    bm_e = 256
    i_out, i_cl_out = pl.pallas_call(
        _stage_item_final,
        grid=(_N_ITEMS // bm_e,),
        in_specs=[
            pl.BlockSpec((bm_e, _N_USERS), _row(bm_e)),
            pl.BlockSpec((_N_USERS, _D), _full()),
            pl.BlockSpec((bm_e, _D), _row(bm_e)),
            pl.BlockSpec((bm_e, _D), _row(bm_e)),
            pl.BlockSpec((bm_e, _D), _row(bm_e)),
            pl.BlockSpec((bm_e, _D), _row(bm_e)),
        ],
        out_specs=[
            pl.BlockSpec((bm_e, _D), _row(bm_e)),
            pl.BlockSpec((bm_e, _D), _row(bm_e)),
        ],
        out_shape=[
            jax.ShapeDtypeStruct((_N_ITEMS, _D), _F32),
            jax.ShapeDtypeStruct((_N_ITEMS, _D), _F32),
        ],
    )(iu_graph, u_g2, item_emb, image_item_feats, text_item_feats, i1n)

    prompt_user = jnp.zeros((_N_USERS, _D), dtype=_F32)
    prompt_item = jnp.zeros((_N_ITEMS, _D), dtype=_F32)
    gcl_loss = jnp.float32(0.0)

    return (u_out, i_out, image_item_feats, text_item_feats,
            image_user_feats, text_user_feats, u_out, i_out,
            prompt_user, prompt_item, gcl_loss)
